# Initial kernel scaffold; baseline (speedup 1.0000x reference)
import jax
import jax.numpy as jnp
from jax.experimental import pallas as pl


def kernel(x, gamma1, gamma2, beta1, beta2, r, edge_index, node_degrees, node_types):
    raise NotImplementedError("write your pallas kernel here")


# speedup vs baseline: 4.0632x; 4.0632x over previous
import functools

import jax
import jax.numpy as jnp
import numpy as np
from jax import lax
from jax.experimental import pallas as pl
from jax.experimental.pallas import tpu as pltpu
from jax.experimental.pallas import tpu_sc as plsc

N = 10000
D = 128
TAIL_K = 5
NC = 2
NS = 16
NW = NC * NS
K = 128
IB = 8
NPAD = 10240
ROWS_PER_SUB = NPAD // NS
F32 = jnp.float32


def _seg_body(table, dsti, srci, out_feat, acc, idxd, idxs, rows, sem):
    c = lax.axis_index("c")
    s = lax.axis_index("s")
    wid = c * NS + s
    nchunk = dsti.shape[1]

    def zb(i, _):
        for j in range(D // 16):
            rows[i, pl.ds(j * 16, 16)] = jnp.zeros((16,), F32)
        return jnp.int32(0)
    lax.fori_loop(jnp.int32(0), jnp.int32(K), zb, jnp.int32(0))

    for j in range(ROWS_PER_SUB // K):
        off = s * ROWS_PER_SUB + j * K
        pltpu.sync_copy(rows, acc.at[pl.ds(off, K)])
    plsc.subcore_barrier()

    def outer(ob, _):
        pltpu.sync_copy(dsti.at[wid, pl.ds(ob * IB, IB)], idxd)
        pltpu.sync_copy(srci.at[wid, pl.ds(ob * IB, IB)], idxs)

        def chunk(ci, _):
            pltpu.async_copy(table.at[idxd.at[ci]], rows, sem).wait()
            pltpu.sync_copy(rows, acc.at[idxs.at[ci]], add=True)
            return jnp.int32(0)
        lax.fori_loop(jnp.int32(0), jnp.int32(IB), chunk, jnp.int32(0))
        return jnp.int32(0)
    lax.fori_loop(jnp.int32(0), jnp.int32(nchunk // IB), outer, jnp.int32(0))

    plsc.subcore_barrier()
    for j in range(ROWS_PER_SUB // K):
        off = s * ROWS_PER_SUB + j * K
        pltpu.sync_copy(acc.at[pl.ds(off, K)], out_feat.at[c, pl.ds(off, K)])


def _deg_body(srci, out_deg, dacc, idxs, ones, zvec, sem):
    del sem
    c = lax.axis_index("c")
    s = lax.axis_index("s")
    wid = c * NS + s
    nchunk = srci.shape[1]

    def fill(i, _):
        for j in range(D // 16):
            ones[i, pl.ds(j * 16, 16)] = jnp.ones((16,), F32)
            zvec[i, pl.ds(j * 16, 16)] = jnp.zeros((16,), F32)
        return jnp.int32(0)
    lax.fori_loop(jnp.int32(0), jnp.int32(K), fill, jnp.int32(0))
    for j in range(ROWS_PER_SUB // K):
        off = s * ROWS_PER_SUB + j * K
        pltpu.sync_copy(zvec, dacc.at[pl.ds(off, K)])
    plsc.subcore_barrier()

    def outer(obk, _):
        pltpu.sync_copy(srci.at[wid, pl.ds(obk * IB, IB)], idxs)

        def chunk(ci, _):
            pltpu.sync_copy(ones, dacc.at[idxs.at[ci]], add=True)
            return jnp.int32(0)
        lax.fori_loop(jnp.int32(0), jnp.int32(IB), chunk, jnp.int32(0))
        return jnp.int32(0)
    lax.fori_loop(jnp.int32(0), jnp.int32(nchunk // IB), outer, jnp.int32(0))

    plsc.subcore_barrier()
    for j in range(ROWS_PER_SUB // K):
        off = s * ROWS_PER_SUB + j * K
        pltpu.sync_copy(dacc.at[pl.ds(off, K)], out_deg.at[c, pl.ds(off, K)])


def _make_seg_kernel():
    mesh = plsc.VectorSubcoreMesh(core_axis_name="c", subcore_axis_name="s")
    return pl.kernel(
        _seg_body,
        out_type=jax.ShapeDtypeStruct((NC, NPAD, D), F32),
        mesh=mesh,
        scratch_types=[
            pltpu.VMEM_SHARED((NPAD, D), F32),
            pltpu.VMEM((IB, K), jnp.int32),
            pltpu.VMEM((IB, K), jnp.int32),
            pltpu.VMEM((K, D), F32),
            pltpu.SemaphoreType.DMA,
        ],
    )


def _make_deg_kernel():
    mesh = plsc.VectorSubcoreMesh(core_axis_name="c", subcore_axis_name="s")
    return pl.kernel(
        _deg_body,
        out_type=jax.ShapeDtypeStruct((NC, NPAD, D), F32),
        mesh=mesh,
        scratch_types=[
            pltpu.VMEM_SHARED((NPAD, D), F32),
            pltpu.VMEM((IB, K), jnp.int32),
            pltpu.VMEM((K, D), F32),
            pltpu.VMEM((K, D), F32),
            pltpu.SemaphoreType.DMA,
        ],
    )


def _leaky(t):
    return jnp.where(t >= 0.0, t, 0.2 * t)


def _tc1_body(x_ref, fp_ref, dp_ref, g1_ref, g2_ref, b1_ref, b2_ref, r_ref,
              hh1_ref, ht1_ref):
    x = x_ref[...]
    raw0 = fp_ref[0] + fp_ref[1]
    deg = (dp_ref[0] + dp_ref[1])[:, 0:1]
    inv_deg = 1.0 / jnp.maximum(deg, 1.0)
    nb = raw0 * inv_deg
    ga = (jnp.dot(x, g1_ref[...], preferred_element_type=F32)
          + jnp.dot(nb, g2_ref[...], preferred_element_type=F32))
    be = (jnp.dot(x, b1_ref[...], preferred_element_type=F32)
          + jnp.dot(nb, b2_ref[...], preferred_element_type=F32))
    m = x + (_leaky(ga) + 1.0) * r_ref[...] + _leaky(be) - nb
    hh1_ref[...] = (raw0 + x) / (deg + 1.0)
    ht1_ref[...] = (raw0 + x + m) / (deg + 2.0)


def _tc2_body(x_ref, hh1_ref, ht1_ref, fph_ref, fpt_ref, dp_ref,
              g1_ref, g2_ref, b1_ref, b2_ref, r_ref, hm_ref, out_ref):
    x = x_ref[...]
    hh1 = hh1_ref[...]
    ht1 = ht1_ref[...]
    deg = (dp_ref[0] + dp_ref[1])[:, 0:1]
    inv_deg = 1.0 / jnp.maximum(deg, 1.0)
    raw1h = fph_ref[0] + fph_ref[1]
    raw1t = fpt_ref[0] + fpt_ref[1]
    hh2 = (raw1h + hh1) / (deg + 1.0)
    nb = raw1t * inv_deg
    ga = (jnp.dot(ht1, g1_ref[...], preferred_element_type=F32)
          + jnp.dot(nb, g2_ref[...], preferred_element_type=F32))
    be = (jnp.dot(ht1, b1_ref[...], preferred_element_type=F32)
          + jnp.dot(nb, b2_ref[...], preferred_element_type=F32))
    m = ht1 + (_leaky(ga) + 1.0) * r_ref[...] + _leaky(be) - nb
    ht2 = (raw1t + ht1 + m) / (deg + 2.0)
    emb_h = (x + hh1 + hh2) / 3.0
    emb_t = (x + ht1 + ht2) / 3.0
    hm = hm_ref[...][:, 0:1]
    out_ref[...] = emb_t + hm * (emb_h - emb_t)


BN = 400
GRID = N // BN

_I0 = np.int32(0)
_row = pl.BlockSpec((BN, D), lambda i: (i, _I0))
_parts = pl.BlockSpec((NC, BN, D), lambda i: (_I0, i, _I0))
_dparts = pl.BlockSpec((NC, BN, 16), lambda i: (_I0, i, _I0))
_wmat = pl.BlockSpec((D, D), lambda i: (_I0, _I0))
_rvec = pl.BlockSpec((1, D), lambda i: (_I0, _I0))
_hmspec = pl.BlockSpec((BN, 16), lambda i: (i, _I0))

_tc1 = pl.pallas_call(
    _tc1_body,
    grid=(GRID,),
    in_specs=[_row, _parts, _parts, _wmat, _wmat, _wmat, _wmat, _rvec],
    out_specs=[_row, _row],
    out_shape=[jax.ShapeDtypeStruct((N, D), F32),
               jax.ShapeDtypeStruct((N, D), F32)],
)

_tc2 = pl.pallas_call(
    _tc2_body,
    grid=(GRID,),
    in_specs=[_row, _row, _row, _parts, _parts, _parts,
              _wmat, _wmat, _wmat, _wmat, _rvec, _hmspec],
    out_specs=_row,
    out_shape=jax.ShapeDtypeStruct((N, D), F32),
)


def kernel(x, gamma1, gamma2, beta1, beta2, r, edge_index, node_degrees,
           node_types):
    x = x.astype(F32)
    E = edge_index.shape[1]
    nchunk = -(-E // (NW * K * IB)) * IB
    epad = NW * nchunk * K
    src = edge_index[0].astype(jnp.int32)
    dst = edge_index[1].astype(jnp.int32)
    src_eff = jnp.where(src == dst, jnp.int32(N), src)
    srcp = jnp.full((epad,), N, jnp.int32).at[:E].set(src_eff)
    srcp = srcp.reshape(NW, nchunk, K)
    dstp = jnp.zeros((epad,), jnp.int32).at[:E].set(dst)
    dstp = dstp.reshape(NW, nchunk, K)

    seg = _make_seg_kernel()
    degk = _make_deg_kernel()

    feat0 = seg(x, dstp, srcp)
    degp = degk(srcp)
    hm = ((node_degrees > TAIL_K) | (node_types != 0)).astype(F32)
    hm16 = jnp.broadcast_to(hm[:, None], (N, 16))

    g10, g20, b10, b20 = (gamma1[0].T, gamma2[0].T, beta1[0].T, beta2[0].T)
    g11, g21, b11, b21 = (gamma1[1].T, gamma2[1].T, beta1[1].T, beta2[1].T)
    hh1, ht1 = _tc1(x, feat0, degp, g10, g20, b10, b20, r[0])
    feath = seg(hh1, dstp, srcp)
    featt = seg(ht1, dstp, srcp)
    out = _tc2(x, hh1, ht1, feath, featt, degp,
               g11, g21, b11, b21, r[1], hm16)
    return out


# speedup vs baseline: 4.4563x; 1.0968x over previous
import functools

import jax
import jax.numpy as jnp
import numpy as np
from jax import lax
from jax.experimental import pallas as pl
from jax.experimental.pallas import tpu as pltpu
from jax.experimental.pallas import tpu_sc as plsc

N = 10000
D = 128
TAIL_K = 5
NC = 2
NS = 16
NW = NC * NS
K = 128
IB = 8
NPAD = 10240
ROWS_PER_SUB = NPAD // NS
F32 = jnp.float32


def _seg_body(table, dsti, srci, out_feat, acc, idxd, idxs, rows_a, rows_b,
              sem_a, sem_b):
    c = lax.axis_index("c")
    s = lax.axis_index("s")
    wid = c * NS + s
    nchunk = dsti.shape[1]

    def zb(i, _):
        for j in range(D // 16):
            rows_a[i, pl.ds(j * 16, 16)] = jnp.zeros((16,), F32)
        return jnp.int32(0)
    lax.fori_loop(jnp.int32(0), jnp.int32(K), zb, jnp.int32(0))

    for j in range(ROWS_PER_SUB // K):
        off = s * ROWS_PER_SUB + j * K
        pltpu.sync_copy(rows_a, acc.at[pl.ds(off, K)])
    plsc.subcore_barrier()

    bufs = (rows_a, rows_b)
    sems = (sem_a, sem_b)

    def outer(ob, _):
        pltpu.sync_copy(dsti.at[wid, pl.ds(ob * IB, IB)], idxd)
        pltpu.sync_copy(srci.at[wid, pl.ds(ob * IB, IB)], idxs)
        cps = [None, None]
        cps[0] = pltpu.async_copy(table.at[idxd.at[np.int32(0)]],
                                  bufs[0], sems[0])
        for ci in range(IB):
            cur = ci % 2
            nxt = (ci + 1) % 2
            if ci + 1 < IB:
                cps[nxt] = pltpu.async_copy(
                    table.at[idxd.at[np.int32(ci + 1)]], bufs[nxt], sems[nxt])
            cps[cur].wait()
            pltpu.sync_copy(bufs[cur], acc.at[idxs.at[np.int32(ci)]],
                            add=True)
        return jnp.int32(0)
    lax.fori_loop(jnp.int32(0), jnp.int32(nchunk // IB), outer, jnp.int32(0))

    plsc.subcore_barrier()
    for j in range(ROWS_PER_SUB // K):
        off = s * ROWS_PER_SUB + j * K
        pltpu.sync_copy(acc.at[pl.ds(off, K)], out_feat.at[c, pl.ds(off, K)])


def _deg_body(srci, out_deg, dacc, idxs, ones, zvec, sem):
    del sem
    c = lax.axis_index("c")
    s = lax.axis_index("s")
    wid = c * NS + s
    nchunk = srci.shape[1]

    def fill(i, _):
        for j in range(D // 16):
            ones[i, pl.ds(j * 16, 16)] = jnp.ones((16,), F32)
            zvec[i, pl.ds(j * 16, 16)] = jnp.zeros((16,), F32)
        return jnp.int32(0)
    lax.fori_loop(jnp.int32(0), jnp.int32(K), fill, jnp.int32(0))
    for j in range(ROWS_PER_SUB // K):
        off = s * ROWS_PER_SUB + j * K
        pltpu.sync_copy(zvec, dacc.at[pl.ds(off, K)])
    plsc.subcore_barrier()

    def outer(obk, _):
        pltpu.sync_copy(srci.at[wid, pl.ds(obk * IB, IB)], idxs)

        def chunk(ci, _):
            pltpu.sync_copy(ones, dacc.at[idxs.at[ci]], add=True)
            return jnp.int32(0)
        lax.fori_loop(jnp.int32(0), jnp.int32(IB), chunk, jnp.int32(0))
        return jnp.int32(0)
    lax.fori_loop(jnp.int32(0), jnp.int32(nchunk // IB), outer, jnp.int32(0))

    plsc.subcore_barrier()
    for j in range(ROWS_PER_SUB // K):
        off = s * ROWS_PER_SUB + j * K
        pltpu.sync_copy(dacc.at[pl.ds(off, K)], out_deg.at[c, pl.ds(off, K)])


def _make_seg_kernel():
    mesh = plsc.VectorSubcoreMesh(core_axis_name="c", subcore_axis_name="s")
    return pl.kernel(
        _seg_body,
        out_type=jax.ShapeDtypeStruct((NC, NPAD, D), F32),
        mesh=mesh,
        scratch_types=[
            pltpu.VMEM_SHARED((NPAD, D), F32),
            pltpu.VMEM((IB, K), jnp.int32),
            pltpu.VMEM((IB, K), jnp.int32),
            pltpu.VMEM((K, D), F32),
            pltpu.VMEM((K, D), F32),
            pltpu.SemaphoreType.DMA,
            pltpu.SemaphoreType.DMA,
        ],
    )


def _make_deg_kernel():
    mesh = plsc.VectorSubcoreMesh(core_axis_name="c", subcore_axis_name="s")
    return pl.kernel(
        _deg_body,
        out_type=jax.ShapeDtypeStruct((NC, NPAD, D), F32),
        mesh=mesh,
        scratch_types=[
            pltpu.VMEM_SHARED((NPAD, D), F32),
            pltpu.VMEM((IB, K), jnp.int32),
            pltpu.VMEM((K, D), F32),
            pltpu.VMEM((K, D), F32),
            pltpu.SemaphoreType.DMA,
        ],
    )


def _leaky(t):
    return jnp.where(t >= 0.0, t, 0.2 * t)


def _tc1_body(x_ref, fp_ref, dp_ref, g1_ref, g2_ref, b1_ref, b2_ref, r_ref,
              hh1_ref, ht1_ref):
    x = x_ref[...]
    raw0 = fp_ref[0] + fp_ref[1]
    deg = (dp_ref[0] + dp_ref[1])[:, 0:1]
    inv_deg = 1.0 / jnp.maximum(deg, 1.0)
    nb = raw0 * inv_deg
    ga = (jnp.dot(x, g1_ref[...], preferred_element_type=F32)
          + jnp.dot(nb, g2_ref[...], preferred_element_type=F32))
    be = (jnp.dot(x, b1_ref[...], preferred_element_type=F32)
          + jnp.dot(nb, b2_ref[...], preferred_element_type=F32))
    m = x + (_leaky(ga) + 1.0) * r_ref[...] + _leaky(be) - nb
    hh1_ref[...] = (raw0 + x) / (deg + 1.0)
    ht1_ref[...] = (raw0 + x + m) / (deg + 2.0)


def _tc2_body(x_ref, hh1_ref, ht1_ref, fph_ref, fpt_ref, dp_ref,
              g1_ref, g2_ref, b1_ref, b2_ref, r_ref, hm_ref, out_ref):
    x = x_ref[...]
    hh1 = hh1_ref[...]
    ht1 = ht1_ref[...]
    deg = (dp_ref[0] + dp_ref[1])[:, 0:1]
    inv_deg = 1.0 / jnp.maximum(deg, 1.0)
    raw1h = fph_ref[0] + fph_ref[1]
    raw1t = fpt_ref[0] + fpt_ref[1]
    hh2 = (raw1h + hh1) / (deg + 1.0)
    nb = raw1t * inv_deg
    ga = (jnp.dot(ht1, g1_ref[...], preferred_element_type=F32)
          + jnp.dot(nb, g2_ref[...], preferred_element_type=F32))
    be = (jnp.dot(ht1, b1_ref[...], preferred_element_type=F32)
          + jnp.dot(nb, b2_ref[...], preferred_element_type=F32))
    m = ht1 + (_leaky(ga) + 1.0) * r_ref[...] + _leaky(be) - nb
    ht2 = (raw1t + ht1 + m) / (deg + 2.0)
    emb_h = (x + hh1 + hh2) / 3.0
    emb_t = (x + ht1 + ht2) / 3.0
    hm = hm_ref[...][:, 0:1]
    out_ref[...] = emb_t + hm * (emb_h - emb_t)


BN = 400
GRID = N // BN

_I0 = np.int32(0)
_row = pl.BlockSpec((BN, D), lambda i: (i, _I0))
_parts = pl.BlockSpec((NC, BN, D), lambda i: (_I0, i, _I0))
_dparts = pl.BlockSpec((NC, BN, 16), lambda i: (_I0, i, _I0))
_wmat = pl.BlockSpec((D, D), lambda i: (_I0, _I0))
_rvec = pl.BlockSpec((1, D), lambda i: (_I0, _I0))
_hmspec = pl.BlockSpec((BN, 16), lambda i: (i, _I0))

_tc1 = pl.pallas_call(
    _tc1_body,
    grid=(GRID,),
    in_specs=[_row, _parts, _parts, _wmat, _wmat, _wmat, _wmat, _rvec],
    out_specs=[_row, _row],
    out_shape=[jax.ShapeDtypeStruct((N, D), F32),
               jax.ShapeDtypeStruct((N, D), F32)],
)

_tc2 = pl.pallas_call(
    _tc2_body,
    grid=(GRID,),
    in_specs=[_row, _row, _row, _parts, _parts, _parts,
              _wmat, _wmat, _wmat, _wmat, _rvec, _hmspec],
    out_specs=_row,
    out_shape=jax.ShapeDtypeStruct((N, D), F32),
)


def kernel(x, gamma1, gamma2, beta1, beta2, r, edge_index, node_degrees,
           node_types):
    x = x.astype(F32)
    E = edge_index.shape[1]
    nchunk = -(-E // (NW * K * IB)) * IB
    epad = NW * nchunk * K
    src = edge_index[0].astype(jnp.int32)
    dst = edge_index[1].astype(jnp.int32)
    src_eff = jnp.where(src == dst, jnp.int32(N), src)
    srcp = jnp.full((epad,), N, jnp.int32).at[:E].set(src_eff)
    srcp = srcp.reshape(NW, nchunk, K)
    dstp = jnp.zeros((epad,), jnp.int32).at[:E].set(dst)
    dstp = dstp.reshape(NW, nchunk, K)

    seg = _make_seg_kernel()
    degk = _make_deg_kernel()

    feat0 = seg(x, dstp, srcp)
    degp = degk(srcp)
    hm = ((node_degrees > TAIL_K) | (node_types != 0)).astype(F32)
    hm16 = jnp.broadcast_to(hm[:, None], (N, 16))

    g10, g20, b10, b20 = (gamma1[0].T, gamma2[0].T, beta1[0].T, beta2[0].T)
    g11, g21, b11, b21 = (gamma1[1].T, gamma2[1].T, beta1[1].T, beta2[1].T)
    hh1, ht1 = _tc1(x, feat0, degp, g10, g20, b10, b20, r[0])
    feath = seg(hh1, dstp, srcp)
    featt = seg(ht1, dstp, srcp)
    out = _tc2(x, hh1, ht1, feath, featt, degp,
               g11, g21, b11, b21, r[1], hm16)
    return out
